# baseline (speedup 1.0000x reference)
import functools

import jax
import jax.numpy as jnp
from jax import lax
from jax.experimental import pallas as pl
from jax.experimental.pallas import tpu as pltpu
from jax.experimental.pallas import tpu_sc as plsc

N = 10000
E = 320000
D_IN = 128
D_HID = 64
D_OUT = 16

NC, NS, L = 2, 16, 16
NT = NC * NS
CH = 128
GR = CH // L
NCHUNK = E // CH
NP = 10240
NR = NP // 16
NA = 2 * N
PT2 = NA // NS
BROW = 1000
GRID = N // BROW

_f32 = jnp.float32
_i32 = jnp.int32


def _pre_body(x_ref, w_ref, asrc_ref, adst_ref,
              h_ref, als_ref, ald_ref, cs_ref, cd_ref, hw_ref):
    b = pl.program_id(0)
    i = pl.program_id(1)
    h = jnp.dot(x_ref[...], w_ref[...], preferred_element_type=_f32)
    h_ref[...] = h
    als = jnp.dot(h, asrc_ref[...], preferred_element_type=_f32)
    ald = jnp.dot(h, adst_ref[...], preferred_element_type=_f32)
    als_ref[...] = als
    ald_ref[...] = ald
    bs = jnp.full((1, 1), jnp.max(als), _f32)
    bd = jnp.full((1, 1), jnp.max(ald), _f32)

    @pl.when(jnp.logical_and(b == 0, i == 0))
    def _():
        cs_ref[...] = bs
        cd_ref[...] = bd

    @pl.when(jnp.logical_and(b == 0, i != 0))
    def _():
        cs_ref[...] = jnp.maximum(cs_ref[...], bs)
        cd_ref[...] = jnp.maximum(cd_ref[...], bd)

    chalf = 0.5 * (cs_ref[0, 0] + cd_ref[0, 0])
    w = jnp.where(b == 1, jnp.exp(als - chalf), jnp.exp(0.2 * als - chalf))
    hw_ref[...] = w * h


def _hw_map(b, i):
    return (jnp.where(b == 0, 2 * GRID, (b - 1) * GRID + i), 0)


def _pre(x, W, asrc, adst):
    d_in, d = W.shape
    return pl.pallas_call(
        _pre_body,
        grid=(3, GRID),
        in_specs=[
            pl.BlockSpec((BROW, d_in), lambda b, i: (i, 0)),
            pl.BlockSpec((d_in, d), lambda b, i: (0, 0)),
            pl.BlockSpec((d, 1), lambda b, i: (0, 0)),
            pl.BlockSpec((d, 1), lambda b, i: (0, 0)),
        ],
        out_specs=[
            pl.BlockSpec((BROW, d), lambda b, i: (i, 0)),
            pl.BlockSpec((BROW, 1), lambda b, i: (i, 0)),
            pl.BlockSpec((BROW, 1), lambda b, i: (i, 0)),
            pl.BlockSpec((1, 1), lambda b, i: (0, 0)),
            pl.BlockSpec((1, 1), lambda b, i: (0, 0)),
            pl.BlockSpec((BROW, d), _hw_map),
        ],
        out_shape=[
            jax.ShapeDtypeStruct((N, d), _f32),
            jax.ShapeDtypeStruct((N, 1), _f32),
            jax.ShapeDtypeStruct((N, 1), _f32),
            jax.ShapeDtypeStruct((1, 1), _f32),
            jax.ShapeDtypeStruct((1, 1), _f32),
            jax.ShapeDtypeStruct((NA + BROW, d), _f32),
        ],
    )(x, W, asrc, adst)


NB = N // BROW


def _norm1pre2_body(acc_p0, acc_n0, acc_p1, acc_n1, den0_ref, den1_ref,
                    h1_ref, als_ref, ald_ref, cs_ref, cd_ref,
                    w2_ref, asrc2_ref, adst2_ref, b1_ref,
                    h2_ref, als2_ref, ald2_ref, cs2_ref, cd2_ref, hw_ref):
    b = pl.program_id(0)
    i = pl.program_id(1)
    c1 = cs_ref[...] + cd_ref[...]
    chalf = 0.5 * c1
    ald = ald_ref[...]
    fp = jnp.exp(ald - chalf)
    fn = jnp.exp(0.2 * ald - chalf)
    v = als_ref[...] + ald
    exl = jnp.exp(jnp.where(v >= 0.0, v, 0.2 * v) - c1)
    num = (fp * (acc_p0[0] + acc_p1[0])
           + fn * (acc_n0[0] + acc_n1[0]) + exl * h1_ref[...])
    den = den0_ref[0] + den1_ref[0] + exl
    z = num / den + b1_ref[...]
    z = jnp.maximum(z, 0.0)
    h2 = jnp.dot(z, w2_ref[...], preferred_element_type=_f32)
    h2_ref[...] = h2
    als2 = jnp.dot(h2, asrc2_ref[...], preferred_element_type=_f32)
    ald2 = jnp.dot(h2, adst2_ref[...], preferred_element_type=_f32)
    als2_ref[...] = als2
    ald2_ref[...] = ald2
    bs = jnp.full((1, 1), jnp.max(als2), _f32)
    bd = jnp.full((1, 1), jnp.max(ald2), _f32)

    @pl.when(jnp.logical_and(b == 0, i == 0))
    def _():
        cs2_ref[...] = bs
        cd2_ref[...] = bd

    @pl.when(jnp.logical_and(b == 0, i != 0))
    def _():
        cs2_ref[...] = jnp.maximum(cs2_ref[...], bs)
        cd2_ref[...] = jnp.maximum(cd2_ref[...], bd)

    chalf2 = 0.5 * (cs2_ref[0, 0] + cd2_ref[0, 0])
    w = jnp.where(b == 1, jnp.exp(als2 - chalf2),
                  jnp.exp(0.2 * als2 - chalf2))
    hw_ref[...] = w * h2


def _norm1pre2(acc, den, h1, als, ald, cs, cd, W2, asrc2, adst2, b1):
    blk = lambda r, c: pl.BlockSpec((r, c), lambda b, i: (i, 0))
    cst = lambda r, c: pl.BlockSpec((r, c), lambda b, i: (0, 0))
    a3 = lambda cid, off, c: pl.BlockSpec(
        (1, BROW, c), lambda b, i: (cid, off + i, 0))
    return pl.pallas_call(
        _norm1pre2_body,
        grid=(3, GRID),
        in_specs=[
            a3(0, 0, D_HID), a3(0, NB, D_HID), a3(1, 0, D_HID),
            a3(1, NB, D_HID), a3(0, 0, 1), a3(1, 0, 1),
            blk(BROW, D_HID), blk(BROW, 1), blk(BROW, 1),
            cst(1, 1), cst(1, 1),
            cst(D_HID, D_OUT), cst(D_OUT, 1), cst(D_OUT, 1), cst(1, D_HID),
        ],
        out_specs=[
            blk(BROW, D_OUT), blk(BROW, 1), blk(BROW, 1),
            cst(1, 1), cst(1, 1),
            pl.BlockSpec((BROW, D_OUT), _hw_map),
        ],
        out_shape=[
            jax.ShapeDtypeStruct((N, D_OUT), _f32),
            jax.ShapeDtypeStruct((N, 1), _f32),
            jax.ShapeDtypeStruct((N, 1), _f32),
            jax.ShapeDtypeStruct((1, 1), _f32),
            jax.ShapeDtypeStruct((1, 1), _f32),
            jax.ShapeDtypeStruct((NA + BROW, D_OUT), _f32),
        ],
    )(acc, acc, acc, acc, den, den, h1, als, ald, cs, cd,
      W2, asrc2, adst2, b1)


def _norm2_body(acc_p0, acc_n0, acc_p1, acc_n1, den0_ref, den1_ref,
                h2_ref, als_ref, ald_ref, cs_ref, cd_ref, b2_ref, z2_ref):
    c2 = cs_ref[...] + cd_ref[...]
    chalf = 0.5 * c2
    ald = ald_ref[...]
    fp = jnp.exp(ald - chalf)
    fn = jnp.exp(0.2 * ald - chalf)
    v = als_ref[...] + ald
    exl = jnp.exp(jnp.where(v >= 0.0, v, 0.2 * v) - c2)
    num = (fp * (acc_p0[0] + acc_p1[0])
           + fn * (acc_n0[0] + acc_n1[0]) + exl * h2_ref[...])
    den = den0_ref[0] + den1_ref[0] + exl
    z2_ref[...] = num / den + b2_ref[...]


def _norm2(acc, den, h2, als, ald, cs, cd, b2):
    blk = lambda r, c: pl.BlockSpec((r, c), lambda i: (i, 0))
    cst = lambda r, c: pl.BlockSpec((r, c), lambda i: (0, 0))
    a3 = lambda cid, off, c: pl.BlockSpec(
        (1, BROW, c), lambda i: (cid, off + i, 0))
    return pl.pallas_call(
        _norm2_body,
        grid=(GRID,),
        in_specs=[
            a3(0, 0, D_OUT), a3(0, NB, D_OUT), a3(1, 0, D_OUT),
            a3(1, NB, D_OUT), a3(0, 0, 1), a3(1, 0, 1),
            blk(BROW, D_OUT), blk(BROW, 1), blk(BROW, 1),
            cst(1, 1), cst(1, 1), cst(1, D_OUT),
        ],
        out_specs=blk(BROW, D_OUT),
        out_shape=jax.ShapeDtypeStruct((N, D_OUT), _f32),
    )(acc, acc, acc, acc, den, den, h2, als, ald, cs, cd, b2)


def _sc_mesh():
    return plsc.VectorSubcoreMesh(
        core_axis_name="c", subcore_axis_name="s",
        num_cores=NC, num_subcores=NS)


def _make_edge_kernel(D):
    NJ = 80

    def body(src_hbm, dst_hbm, hw_hbm, als_hbm, ald_hbm, cvec_hbm,
             zmat_hbm, zden_hbm, iident_hbm,
             acc_hbm, den_hbm,
             als_t, ald_t, den_t,
             sidx0, sidx1, didx0, didx1, gidx0, gidx1, sdidx0, sdidx1,
             rows0, rows1, iidx, cvec_t,
             acc_s, den_s,
             sem_i0, sem_i1, sem_g0, sem_g1, sem_s0, sem_s1):
        cid = lax.axis_index("c")
        sid = lax.axis_index("s")
        wid = sid * NC + cid
        sidx = (sidx0, sidx1)
        didx = (didx0, didx1)
        gidx = (gidx0, gidx1)
        sdidx = (sdidx0, sdidx1)
        rows = (rows0, rows1)
        sem_i = (sem_i0, sem_i1)
        sem_g = (sem_g0, sem_g1)
        sem_s = (sem_s0, sem_s1)

        pltpu.sync_copy(als_hbm, als_t)
        pltpu.sync_copy(ald_hbm, ald_t)
        pltpu.sync_copy(iident_hbm, iidx)
        pltpu.sync_copy(zden_hbm, den_t)
        pltpu.sync_copy(zmat_hbm, acc_s.at[pl.ds(sid * PT2, PT2)])
        pltpu.sync_copy(zden_hbm.at[pl.ds(sid * 40, 40)],
                        den_s.at[pl.ds(sid * 40, 40)])
        pltpu.sync_copy(cvec_hbm, cvec_t)
        c = cvec_t[...]

        plsc.subcore_barrier()

        zvec = jnp.zeros((L,), _i32)
        nvec = jnp.full((L,), N, _i32)

        def _idx_copies(b, ch):
            off = pl.ds(ch * CH, CH)
            return (pltpu.make_async_copy(src_hbm.at[off], sidx[b], sem_i[b]),
                    pltpu.make_async_copy(dst_hbm.at[off], didx[b], sem_i[b]))

        for b in range(2):
            ch = wid + NT * b

            @pl.when(ch < NCHUNK)
            def _():
                ca, cb = _idx_copies(b, ch)
                ca.start()
                cb.start()

        def _iter(i, _):
            for b in range(2):
                ch = wid + NT * (2 * i + b)

                @pl.when(ch < NCHUNK)
                def _():
                    ca, cb = _idx_copies(b, ch)
                    ca.wait()
                    cb.wait()
                    sct = pltpu.make_async_copy(
                        rows[b], acc_s.at[sdidx[b]], sem_s[b])

                    @pl.when(i > 0)
                    def _():
                        sct.wait()

                    for g in range(GR):
                        sl = pl.ds(g * L, L)
                        sv = sidx[b][sl]
                        dv = didx[b][sl]
                        a1 = plsc.load_gather(als_t, [sv])
                        a2 = plsc.load_gather(ald_t, [dv])
                        e = a1 + a2
                        el = jnp.maximum(e, 0.2 * e)
                        ex = jnp.exp(el - c)
                        plsc.addupdate_scatter(
                            den_t, [lax.shift_right_logical(dv, 4),
                                    lax.bitwise_and(dv, 15)], ex)
                        moff = jnp.where(e < 0.0, nvec, zvec)
                        gidx[b][sl] = sv + moff
                        sdidx[b][sl] = dv + moff
                    gat = pltpu.make_async_copy(hw_hbm.at[gidx[b]],
                                                rows[b], sem_g[b])
                    gat.start()
                    ch2 = ch + 2 * NT

                    @pl.when(ch2 < NCHUNK)
                    def _():
                        na, nb = _idx_copies(b, ch2)
                        na.start()
                        nb.start()

                    gat.wait()
                    sct.start(add=True)
            return 0

        lax.fori_loop(0, NJ // 2, _iter, 0)
        for b in range(2):
            pltpu.make_async_copy(rows[b], acc_s.at[sdidx[b]], sem_s[b]).wait()

        plsc.subcore_barrier()
        pltpu.sync_copy(den_t, den_s.at[iidx], add=True)
        plsc.subcore_barrier()

        pltpu.sync_copy(acc_s.at[pl.ds(sid * PT2, PT2)],
                        acc_hbm.at[cid, pl.ds(sid * PT2, PT2)])

        @pl.when(sid == 0)
        def _():
            pltpu.sync_copy(den_s, den_hbm.at[cid])

    return pl.kernel(
        body,
        out_type=(
            jax.ShapeDtypeStruct((NC, NA, D), _f32),
            jax.ShapeDtypeStruct((NC, NR, 16), _f32),
        ),
        mesh=_sc_mesh(),
        compiler_params=pltpu.CompilerParams(
            needs_layout_passes=False, use_tc_tiling_on_sc=False),
        scratch_types=[
            pltpu.VMEM((N,), _f32),
            pltpu.VMEM((N,), _f32),
            pltpu.VMEM((NR, 16), _f32),
            pltpu.VMEM((CH,), _i32),
            pltpu.VMEM((CH,), _i32),
            pltpu.VMEM((CH,), _i32),
            pltpu.VMEM((CH,), _i32),
            pltpu.VMEM((CH,), _i32),
            pltpu.VMEM((CH,), _i32),
            pltpu.VMEM((CH,), _i32),
            pltpu.VMEM((CH,), _i32),
            pltpu.VMEM((CH, D), _f32),
            pltpu.VMEM((CH, D), _f32),
            pltpu.VMEM((NR,), _i32),
            pltpu.VMEM((L,), _f32),
            pltpu.VMEM_SHARED((NA, D), _f32),
            pltpu.VMEM_SHARED((NR, 16), _f32),
            pltpu.SemaphoreType.DMA,
            pltpu.SemaphoreType.DMA,
            pltpu.SemaphoreType.DMA,
            pltpu.SemaphoreType.DMA,
            pltpu.SemaphoreType.DMA,
            pltpu.SemaphoreType.DMA,
        ],
    )


_edge64 = _make_edge_kernel(D_HID)
_edge16 = _make_edge_kernel(D_OUT)


NJT = 80


def _dec_body(zt_hbm, s_hbm, d_hbm, out_hbm,
              tab, sidx0, sidx1, didx0, didx1, part, obuf0, obuf1,
              sem_i0, sem_i1, sem_o0, sem_o1):
    cid = lax.axis_index("c")
    sid = lax.axis_index("s")
    wid = sid * NC + cid
    sidx = (sidx0, sidx1)
    didx = (didx0, didx1)
    obuf = (obuf0, obuf1)
    sem_i = (sem_i0, sem_i1)
    sem_o = (sem_o0, sem_o1)

    def _idx_copies(b, ch):
        off = pl.ds(ch * CH, CH)
        return (pltpu.make_async_copy(s_hbm.at[off], sidx[b], sem_i[b]),
                pltpu.make_async_copy(d_hbm.at[off], didx[b], sem_i[b]))

    for p in range(2):
        pltpu.sync_copy(zt_hbm.at[pl.ds(8 * p, 8)], tab)

        for b in range(2):
            ch = wid + NT * b

            @pl.when(ch < NCHUNK)
            def _():
                ca, cb = _idx_copies(b, ch)
                ca.start()
                cb.start()

        def _iter(i, _):
            for b in range(2):
                j = 2 * i + b
                ch = wid + NT * j

                @pl.when(ch < NCHUNK)
                def _():
                    ca, cb = _idx_copies(b, ch)
                    ca.wait()
                    cb.wait()
                    ocp = pltpu.make_async_copy(
                        obuf[b], out_hbm.at[pl.ds(ch * CH, CH)], sem_o[b])

                    if p == 1:
                        @pl.when(i > 0)
                        def _():
                            ocp.wait()

                    for g in range(GR):
                        sl = pl.ds(g * L, L)
                        sv = sidx[b][sl]
                        dv = didx[b][sl]
                        acc = jnp.zeros((L,), _f32)
                        for dd in range(8):
                            dsp = jnp.full((L,), dd, _i32)
                            zs = plsc.load_gather(tab, [dsp, sv])
                            zd = plsc.load_gather(tab, [dsp, dv])
                            acc = acc + zs * zd
                        po = pl.ds(j * CH + g * L, L)
                        if p == 0:
                            part[po] = acc
                        else:
                            obuf[b][sl] = acc + part[po]
                    ch2 = ch + 2 * NT

                    @pl.when(ch2 < NCHUNK)
                    def _():
                        na, nb = _idx_copies(b, ch2)
                        na.start()
                        nb.start()

                    if p == 1:
                        ocp.start()
            return 0

        lax.fori_loop(0, NJT // 2, _iter, 0)
        if p == 1:
            for b in range(2):
                ch = wid + NT * b
                pltpu.make_async_copy(
                    obuf[b], out_hbm.at[pl.ds(ch * CH, CH)], sem_o[b]).wait()


_decode = pl.kernel(
    _dec_body,
    out_type=jax.ShapeDtypeStruct((E,), _f32),
    mesh=_sc_mesh(),
    compiler_params=pltpu.CompilerParams(
        needs_layout_passes=False, use_tc_tiling_on_sc=False),
    scratch_types=[
        pltpu.VMEM((8, N), _f32),
        pltpu.VMEM((CH,), _i32),
        pltpu.VMEM((CH,), _i32),
        pltpu.VMEM((CH,), _i32),
        pltpu.VMEM((CH,), _i32),
        pltpu.VMEM((NJT * CH,), _f32),
        pltpu.VMEM((CH,), _f32),
        pltpu.VMEM((CH,), _f32),
        pltpu.SemaphoreType.DMA,
        pltpu.SemaphoreType.DMA,
        pltpu.SemaphoreType.DMA,
        pltpu.SemaphoreType.DMA,
    ],
)


def kernel(x, edge_index, edge_label_index, W1, a_src1, a_dst1, b1,
           W2, a_src2, a_dst2, b2):
    src1d = edge_index[0].astype(_i32)
    dst1d = edge_index[1].astype(_i32)
    eli0 = edge_label_index[0].astype(_i32)
    eli1 = edge_label_index[1].astype(_i32)

    zden = jnp.zeros((NR, 16), _f32)
    iident = jnp.arange(NR, dtype=_i32)

    h1, als1, ald1, cs1, cd1, hw1 = _pre(
        x, W1, a_src1.reshape(D_HID, 1), a_dst1.reshape(D_HID, 1))

    cvec1 = jnp.broadcast_to((cs1 + cd1).reshape(1), (L,))
    acc1, den1 = _edge64(
        src1d, dst1d, hw1, als1.reshape(N), ald1.reshape(N), cvec1,
        jnp.zeros((PT2, D_HID), _f32), zden, iident)

    h2, als2, ald2, cs2, cd2, hw2 = _norm1pre2(
        acc1, den1.reshape(NC, NP, 1),
        h1, als1, ald1, cs1, cd1,
        W2, a_src2.reshape(D_OUT, 1), a_dst2.reshape(D_OUT, 1),
        b1.reshape(1, D_HID))

    cvec2 = jnp.broadcast_to((cs2 + cd2).reshape(1), (L,))
    acc2, den2 = _edge16(
        src1d, dst1d, hw2, als2.reshape(N), ald2.reshape(N), cvec2,
        jnp.zeros((PT2, D_OUT), _f32), zden, iident)

    z2 = _norm2(
        acc2, den2.reshape(NC, NP, 1),
        h2, als2, ald2, cs2, cd2, b2.reshape(1, D_OUT))

    return _decode(z2.T, eli0, eli1)


# speedup vs baseline: 1.0213x; 1.0213x over previous
import functools

import jax
import jax.numpy as jnp
from jax import lax
from jax.experimental import pallas as pl
from jax.experimental.pallas import tpu as pltpu
from jax.experimental.pallas import tpu_sc as plsc

N = 10000
E = 320000
D_IN = 128
D_HID = 64
D_OUT = 16

NC, NS, L = 2, 16, 16
NT = NC * NS
CH = 128
GR = CH // L
NCHUNK = E // CH
NP = 10240
NR = NP // 16
NA = 2 * N
PT2 = NA // NS
BROW = 1000
GRID = N // BROW

_f32 = jnp.float32
_i32 = jnp.int32


def _pre_body(x_ref, w_ref, asrc_ref, adst_ref,
              h_ref, als_ref, ald_ref, cs_ref, cd_ref):
    i = pl.program_id(0)
    h = jnp.dot(x_ref[...], w_ref[...], preferred_element_type=_f32)
    h_ref[...] = h
    als = jnp.dot(h, asrc_ref[...], preferred_element_type=_f32)
    ald = jnp.dot(h, adst_ref[...], preferred_element_type=_f32)
    als_ref[...] = als
    ald_ref[...] = ald
    bs = jnp.full((1, 1), jnp.max(als), _f32)
    bd = jnp.full((1, 1), jnp.max(ald), _f32)

    @pl.when(i == 0)
    def _():
        cs_ref[...] = bs
        cd_ref[...] = bd

    @pl.when(i != 0)
    def _():
        cs_ref[...] = jnp.maximum(cs_ref[...], bs)
        cd_ref[...] = jnp.maximum(cd_ref[...], bd)


def _pre(x, W, asrc, adst):
    d_in, d = W.shape
    return pl.pallas_call(
        _pre_body,
        grid=(GRID,),
        in_specs=[
            pl.BlockSpec((BROW, d_in), lambda i: (i, 0)),
            pl.BlockSpec((d_in, d), lambda i: (0, 0)),
            pl.BlockSpec((d, 1), lambda i: (0, 0)),
            pl.BlockSpec((d, 1), lambda i: (0, 0)),
        ],
        out_specs=[
            pl.BlockSpec((BROW, d), lambda i: (i, 0)),
            pl.BlockSpec((BROW, 1), lambda i: (i, 0)),
            pl.BlockSpec((BROW, 1), lambda i: (i, 0)),
            pl.BlockSpec((1, 1), lambda i: (0, 0)),
            pl.BlockSpec((1, 1), lambda i: (0, 0)),
        ],
        out_shape=[
            jax.ShapeDtypeStruct((N, d), _f32),
            jax.ShapeDtypeStruct((N, 1), _f32),
            jax.ShapeDtypeStruct((N, 1), _f32),
            jax.ShapeDtypeStruct((1, 1), _f32),
            jax.ShapeDtypeStruct((1, 1), _f32),
        ],
    )(x, W, asrc, adst)


def _mkhw_body(h_ref, als_ref, cs_ref, cd_ref, hw_ref):
    b = pl.program_id(0)
    chalf = 0.5 * (cs_ref[0, 0] + cd_ref[0, 0])
    a = als_ref[...]
    w = jnp.where(b == 0, jnp.exp(a - chalf), jnp.exp(0.2 * a - chalf))
    hw_ref[...] = w * h_ref[...]


def _mkhw(h, als, cs, cd):
    d = h.shape[1]
    return pl.pallas_call(
        _mkhw_body,
        grid=(2, GRID),
        in_specs=[
            pl.BlockSpec((BROW, d), lambda b, i: (i, 0)),
            pl.BlockSpec((BROW, 1), lambda b, i: (i, 0)),
            pl.BlockSpec((1, 1), lambda b, i: (0, 0)),
            pl.BlockSpec((1, 1), lambda b, i: (0, 0)),
        ],
        out_specs=pl.BlockSpec((BROW, d), lambda b, i: (b * GRID + i, 0)),
        out_shape=jax.ShapeDtypeStruct((NA, d), _f32),
    )(h, als, cs, cd)


def _norm1pre2_body(pp0_ref, pn0_ref, pp1_ref, pn1_ref, den0_ref, den1_ref,
                    h1_ref, als_ref, ald_ref, cs_ref, cd_ref,
                    w2_ref, asrc2_ref, adst2_ref, b1_ref,
                    h2_ref, als2_ref, ald2_ref, cs2_ref, cd2_ref):
    i = pl.program_id(0)
    c1 = cs_ref[...] + cd_ref[...]
    chalf = 0.5 * c1
    ald = ald_ref[...]
    fp = jnp.exp(ald - chalf)
    fn = jnp.exp(0.2 * ald - chalf)
    v = als_ref[...] + ald
    exl = jnp.exp(jnp.where(v >= 0.0, v, 0.2 * v) - c1)
    num = (fp * (pp0_ref[...] + pp1_ref[...])
           + fn * (pn0_ref[...] + pn1_ref[...]) + exl * h1_ref[...])
    den = den0_ref[...] + den1_ref[...] + exl
    z = num / den + b1_ref[...]
    z = jnp.maximum(z, 0.0)
    h2 = jnp.dot(z, w2_ref[...], preferred_element_type=_f32)
    h2_ref[...] = h2
    als2 = jnp.dot(h2, asrc2_ref[...], preferred_element_type=_f32)
    ald2 = jnp.dot(h2, adst2_ref[...], preferred_element_type=_f32)
    als2_ref[...] = als2
    ald2_ref[...] = ald2
    bs = jnp.full((1, 1), jnp.max(als2), _f32)
    bd = jnp.full((1, 1), jnp.max(ald2), _f32)

    @pl.when(i == 0)
    def _():
        cs2_ref[...] = bs
        cd2_ref[...] = bd

    @pl.when(i != 0)
    def _():
        cs2_ref[...] = jnp.maximum(cs2_ref[...], bs)
        cd2_ref[...] = jnp.maximum(cd2_ref[...], bd)


def _norm1pre2(pp0, pn0, pp1, pn1, den0, den1,
               h1, als, ald, cs, cd, W2, asrc2, adst2, b1):
    blk = lambda r, c: pl.BlockSpec((r, c), lambda i: (i, 0))
    cst = lambda r, c: pl.BlockSpec((r, c), lambda i: (0, 0))
    return pl.pallas_call(
        _norm1pre2_body,
        grid=(GRID,),
        in_specs=[
            blk(BROW, D_HID), blk(BROW, D_HID), blk(BROW, D_HID),
            blk(BROW, D_HID), blk(BROW, 1), blk(BROW, 1),
            blk(BROW, D_HID), blk(BROW, 1), blk(BROW, 1),
            cst(1, 1), cst(1, 1),
            cst(D_HID, D_OUT), cst(D_OUT, 1), cst(D_OUT, 1), cst(1, D_HID),
        ],
        out_specs=[
            blk(BROW, D_OUT), blk(BROW, 1), blk(BROW, 1),
            cst(1, 1), cst(1, 1),
        ],
        out_shape=[
            jax.ShapeDtypeStruct((N, D_OUT), _f32),
            jax.ShapeDtypeStruct((N, 1), _f32),
            jax.ShapeDtypeStruct((N, 1), _f32),
            jax.ShapeDtypeStruct((1, 1), _f32),
            jax.ShapeDtypeStruct((1, 1), _f32),
        ],
    )(pp0, pn0, pp1, pn1, den0, den1, h1, als, ald, cs, cd,
      W2, asrc2, adst2, b1)


def _norm2_body(pp0_ref, pn0_ref, pp1_ref, pn1_ref, den0_ref, den1_ref,
                h2_ref, als_ref, ald_ref, cs_ref, cd_ref, b2_ref, z2_ref):
    c2 = cs_ref[...] + cd_ref[...]
    chalf = 0.5 * c2
    ald = ald_ref[...]
    fp = jnp.exp(ald - chalf)
    fn = jnp.exp(0.2 * ald - chalf)
    v = als_ref[...] + ald
    exl = jnp.exp(jnp.where(v >= 0.0, v, 0.2 * v) - c2)
    num = (fp * (pp0_ref[...] + pp1_ref[...])
           + fn * (pn0_ref[...] + pn1_ref[...]) + exl * h2_ref[...])
    den = den0_ref[...] + den1_ref[...] + exl
    z2_ref[...] = num / den + b2_ref[...]


def _norm2(pp0, pn0, pp1, pn1, den0, den1, h2, als, ald, cs, cd, b2):
    blk = lambda r, c: pl.BlockSpec((r, c), lambda i: (i, 0))
    cst = lambda r, c: pl.BlockSpec((r, c), lambda i: (0, 0))
    return pl.pallas_call(
        _norm2_body,
        grid=(GRID,),
        in_specs=[
            blk(BROW, D_OUT), blk(BROW, D_OUT), blk(BROW, D_OUT),
            blk(BROW, D_OUT), blk(BROW, 1), blk(BROW, 1),
            blk(BROW, D_OUT), blk(BROW, 1), blk(BROW, 1),
            cst(1, 1), cst(1, 1), cst(1, D_OUT),
        ],
        out_specs=blk(BROW, D_OUT),
        out_shape=jax.ShapeDtypeStruct((N, D_OUT), _f32),
    )(pp0, pn0, pp1, pn1, den0, den1, h2, als, ald, cs, cd, b2)


def _sc_mesh():
    return plsc.VectorSubcoreMesh(
        core_axis_name="c", subcore_axis_name="s",
        num_cores=NC, num_subcores=NS)


def _make_edge_kernel(D):
    NJ = 80

    def body(src_hbm, dst_hbm, hw_hbm, als_hbm, ald_hbm, cvec_hbm,
             zmat_hbm, zden_hbm, iident_hbm,
             acc_hbm, den_hbm,
             als_t, ald_t, den_t,
             sidx0, sidx1, didx0, didx1, gidx0, gidx1, sdidx0, sdidx1,
             rows0, rows1, iidx, cvec_t,
             acc_s, den_s,
             sem_i0, sem_i1, sem_g0, sem_g1, sem_s0, sem_s1):
        cid = lax.axis_index("c")
        sid = lax.axis_index("s")
        wid = sid * NC + cid
        sidx = (sidx0, sidx1)
        didx = (didx0, didx1)
        gidx = (gidx0, gidx1)
        sdidx = (sdidx0, sdidx1)
        rows = (rows0, rows1)
        sem_i = (sem_i0, sem_i1)
        sem_g = (sem_g0, sem_g1)
        sem_s = (sem_s0, sem_s1)

        pltpu.sync_copy(als_hbm, als_t)
        pltpu.sync_copy(ald_hbm, ald_t)
        pltpu.sync_copy(iident_hbm, iidx)
        pltpu.sync_copy(zden_hbm, den_t)
        pltpu.sync_copy(zmat_hbm, acc_s.at[pl.ds(sid * PT2, PT2)])
        pltpu.sync_copy(zden_hbm.at[pl.ds(sid * 40, 40)],
                        den_s.at[pl.ds(sid * 40, 40)])
        pltpu.sync_copy(cvec_hbm, cvec_t)
        c = cvec_t[...]

        plsc.subcore_barrier()

        zvec = jnp.zeros((L,), _i32)
        nvec = jnp.full((L,), N, _i32)

        def _idx_copies(b, ch):
            off = pl.ds(ch * CH, CH)
            return (pltpu.make_async_copy(src_hbm.at[off], sidx[b], sem_i[b]),
                    pltpu.make_async_copy(dst_hbm.at[off], didx[b], sem_i[b]))

        for b in range(2):
            ch = wid + NT * b

            @pl.when(ch < NCHUNK)
            def _():
                ca, cb = _idx_copies(b, ch)
                ca.start()
                cb.start()

        def _iter(i, _):
            for b in range(2):
                ch = wid + NT * (2 * i + b)

                @pl.when(ch < NCHUNK)
                def _():
                    ca, cb = _idx_copies(b, ch)
                    ca.wait()
                    cb.wait()
                    sct = pltpu.make_async_copy(
                        rows[b], acc_s.at[sdidx[b]], sem_s[b])

                    @pl.when(i > 0)
                    def _():
                        sct.wait()

                    for g in range(GR):
                        sl = pl.ds(g * L, L)
                        sv = sidx[b][sl]
                        dv = didx[b][sl]
                        a1 = plsc.load_gather(als_t, [sv])
                        a2 = plsc.load_gather(ald_t, [dv])
                        e = a1 + a2
                        el = jnp.where(e >= 0.0, e, 0.2 * e)
                        ex = jnp.exp(el - c)
                        plsc.addupdate_scatter(
                            den_t, [lax.shift_right_logical(dv, 4),
                                    lax.bitwise_and(dv, 15)], ex)
                        moff = jnp.where(e < 0.0, nvec, zvec)
                        gidx[b][sl] = sv + moff
                        sdidx[b][sl] = dv + moff
                    gat = pltpu.make_async_copy(hw_hbm.at[gidx[b]],
                                                rows[b], sem_g[b])
                    gat.start()
                    ch2 = ch + 2 * NT

                    @pl.when(ch2 < NCHUNK)
                    def _():
                        na, nb = _idx_copies(b, ch2)
                        na.start()
                        nb.start()

                    gat.wait()
                    sct.start(add=True)
            return 0

        lax.fori_loop(0, NJ // 2, _iter, 0)
        for b in range(2):
            pltpu.make_async_copy(rows[b], acc_s.at[sdidx[b]], sem_s[b]).wait()

        plsc.subcore_barrier()
        pltpu.sync_copy(den_t, den_s.at[iidx], add=True)
        plsc.subcore_barrier()

        pltpu.sync_copy(acc_s.at[pl.ds(sid * PT2, PT2)],
                        acc_hbm.at[cid, pl.ds(sid * PT2, PT2)])

        @pl.when(sid == 0)
        def _():
            pltpu.sync_copy(den_s, den_hbm.at[cid])

    return pl.kernel(
        body,
        out_type=(
            jax.ShapeDtypeStruct((NC, NA, D), _f32),
            jax.ShapeDtypeStruct((NC, NR, 16), _f32),
        ),
        mesh=_sc_mesh(),
        compiler_params=pltpu.CompilerParams(
            needs_layout_passes=False, use_tc_tiling_on_sc=False),
        scratch_types=[
            pltpu.VMEM((N,), _f32),
            pltpu.VMEM((N,), _f32),
            pltpu.VMEM((NR, 16), _f32),
            pltpu.VMEM((CH,), _i32),
            pltpu.VMEM((CH,), _i32),
            pltpu.VMEM((CH,), _i32),
            pltpu.VMEM((CH,), _i32),
            pltpu.VMEM((CH,), _i32),
            pltpu.VMEM((CH,), _i32),
            pltpu.VMEM((CH,), _i32),
            pltpu.VMEM((CH,), _i32),
            pltpu.VMEM((CH, D), _f32),
            pltpu.VMEM((CH, D), _f32),
            pltpu.VMEM((NR,), _i32),
            pltpu.VMEM((L,), _f32),
            pltpu.VMEM_SHARED((NA, D), _f32),
            pltpu.VMEM_SHARED((NR, 16), _f32),
            pltpu.SemaphoreType.DMA,
            pltpu.SemaphoreType.DMA,
            pltpu.SemaphoreType.DMA,
            pltpu.SemaphoreType.DMA,
            pltpu.SemaphoreType.DMA,
            pltpu.SemaphoreType.DMA,
        ],
    )


_edge64 = _make_edge_kernel(D_HID)
_edge16 = _make_edge_kernel(D_OUT)


NJT = 80


def _dec_body(zt_hbm, s_hbm, d_hbm, out_hbm,
              tab, sidx0, sidx1, didx0, didx1, part, obuf0, obuf1,
              sem_i0, sem_i1, sem_o0, sem_o1):
    cid = lax.axis_index("c")
    sid = lax.axis_index("s")
    wid = sid * NC + cid
    sidx = (sidx0, sidx1)
    didx = (didx0, didx1)
    obuf = (obuf0, obuf1)
    sem_i = (sem_i0, sem_i1)
    sem_o = (sem_o0, sem_o1)

    def _idx_copies(b, ch):
        off = pl.ds(ch * CH, CH)
        return (pltpu.make_async_copy(s_hbm.at[off], sidx[b], sem_i[b]),
                pltpu.make_async_copy(d_hbm.at[off], didx[b], sem_i[b]))

    for p in range(2):
        pltpu.sync_copy(zt_hbm.at[pl.ds(8 * p, 8)], tab)

        for b in range(2):
            ch = wid + NT * b

            @pl.when(ch < NCHUNK)
            def _():
                ca, cb = _idx_copies(b, ch)
                ca.start()
                cb.start()

        def _iter(i, _):
            for b in range(2):
                j = 2 * i + b
                ch = wid + NT * j

                @pl.when(ch < NCHUNK)
                def _():
                    ca, cb = _idx_copies(b, ch)
                    ca.wait()
                    cb.wait()
                    ocp = pltpu.make_async_copy(
                        obuf[b], out_hbm.at[pl.ds(ch * CH, CH)], sem_o[b])

                    if p == 1:
                        @pl.when(i > 0)
                        def _():
                            ocp.wait()

                    for g in range(GR):
                        sl = pl.ds(g * L, L)
                        sv = sidx[b][sl]
                        dv = didx[b][sl]
                        acc = jnp.zeros((L,), _f32)
                        for dd in range(8):
                            dsp = jnp.full((L,), dd, _i32)
                            zs = plsc.load_gather(tab, [dsp, sv])
                            zd = plsc.load_gather(tab, [dsp, dv])
                            acc = acc + zs * zd
                        po = pl.ds(j * CH + g * L, L)
                        if p == 0:
                            part[po] = acc
                        else:
                            obuf[b][sl] = acc + part[po]
                    ch2 = ch + 2 * NT

                    @pl.when(ch2 < NCHUNK)
                    def _():
                        na, nb = _idx_copies(b, ch2)
                        na.start()
                        nb.start()

                    if p == 1:
                        ocp.start()
            return 0

        lax.fori_loop(0, NJT // 2, _iter, 0)
        if p == 1:
            for b in range(2):
                ch = wid + NT * b
                pltpu.make_async_copy(
                    obuf[b], out_hbm.at[pl.ds(ch * CH, CH)], sem_o[b]).wait()


_decode = pl.kernel(
    _dec_body,
    out_type=jax.ShapeDtypeStruct((E,), _f32),
    mesh=_sc_mesh(),
    compiler_params=pltpu.CompilerParams(
        needs_layout_passes=False, use_tc_tiling_on_sc=False),
    scratch_types=[
        pltpu.VMEM((8, N), _f32),
        pltpu.VMEM((CH,), _i32),
        pltpu.VMEM((CH,), _i32),
        pltpu.VMEM((CH,), _i32),
        pltpu.VMEM((CH,), _i32),
        pltpu.VMEM((NJT * CH,), _f32),
        pltpu.VMEM((CH,), _f32),
        pltpu.VMEM((CH,), _f32),
        pltpu.SemaphoreType.DMA,
        pltpu.SemaphoreType.DMA,
        pltpu.SemaphoreType.DMA,
        pltpu.SemaphoreType.DMA,
    ],
)


def kernel(x, edge_index, edge_label_index, W1, a_src1, a_dst1, b1,
           W2, a_src2, a_dst2, b2):
    src1d = edge_index[0].astype(_i32)
    dst1d = edge_index[1].astype(_i32)
    eli0 = edge_label_index[0].astype(_i32)
    eli1 = edge_label_index[1].astype(_i32)

    zden = jnp.zeros((NR, 16), _f32)
    iident = jnp.arange(NR, dtype=_i32)

    h1, als1, ald1, cs1, cd1 = _pre(
        x, W1, a_src1.reshape(D_HID, 1), a_dst1.reshape(D_HID, 1))

    hw1 = _mkhw(h1, als1, cs1, cd1)
    cvec1 = jnp.broadcast_to((cs1 + cd1).reshape(1), (L,))
    acc1, den1 = _edge64(
        src1d, dst1d, hw1, als1.reshape(N), ald1.reshape(N), cvec1,
        jnp.zeros((PT2, D_HID), _f32), zden, iident)

    h2, als2, ald2, cs2, cd2 = _norm1pre2(
        acc1[0, :N], acc1[0, N:], acc1[1, :N], acc1[1, N:],
        den1[0].reshape(NP, 1)[:N], den1[1].reshape(NP, 1)[:N],
        h1, als1, ald1, cs1, cd1,
        W2, a_src2.reshape(D_OUT, 1), a_dst2.reshape(D_OUT, 1),
        b1.reshape(1, D_HID))

    hw2 = _mkhw(h2, als2, cs2, cd2)
    cvec2 = jnp.broadcast_to((cs2 + cd2).reshape(1), (L,))
    acc2, den2 = _edge16(
        src1d, dst1d, hw2, als2.reshape(N), ald2.reshape(N), cvec2,
        jnp.zeros((PT2, D_OUT), _f32), zden, iident)

    z2 = _norm2(
        acc2[0, :N], acc2[0, N:], acc2[1, :N], acc2[1, N:],
        den2[0].reshape(NP, 1)[:N], den2[1].reshape(NP, 1)[:N],
        h2, als2, ald2, cs2, cd2, b2.reshape(1, D_OUT))

    return _decode(z2.T, eli0, eli1)
